# baseline (speedup 1.0000x reference)
import jax
import jax.numpy as jnp
from jax import lax
from jax.experimental import pallas as pl
from jax.experimental.pallas import tpu as pltpu
from jax.experimental.pallas import tpu_sc as plsc

_EMBED_DIM = 32
_BATCH = 4096
_SEQ_LEN = 200

_NC = 2
_NS = 16
_NW = _NC * _NS
_BB = _BATCH // _NW
_LT = _SEQ_LEN // 8
_DT = _EMBED_DIM // 8


def _body(xv_hbm, table_hbm, o5_hbm, idxs, rows_v, t_v, gsem, wsem):
    w = lax.axis_index("s") * _NC + lax.axis_index("c")

    pltpu.sync_copy(xv_hbm.at[:, w], idxs)

    lane = lax.iota(jnp.int32, 16)
    row_idx = [lane + 16 * bg for bg in range(_BB // 16)]
    col_idx = [jnp.full((16,), d, jnp.int32) for d in range(_EMBED_DIM)]

    def fire_gather(l, nb):
        pltpu.async_copy(
            table_hbm.at[idxs.at[lax.div(l, 8), lax.rem(l, 8)]],
            rows_v.at[nb],
            gsem.at[nb],
        )

    def drain_gather(nb):
        pltpu.make_async_copy(
            table_hbm.at[pl.ds(0, _BB)], rows_v.at[nb], gsem.at[nb]
        ).wait()

    def fire_write(l, nb):
        pltpu.async_copy(t_v.at[nb], o5_hbm.at[l].at[:, w], wsem.at[nb])

    def drain_write(nb):
        pltpu.make_async_copy(
            t_v.at[nb], o5_hbm.at[0].at[:, 0], wsem.at[nb]
        ).wait()

    def transpose(nb):
        for dt in range(_DT):
            for di in range(8):
                d = dt * 8 + di
                for bg in range(_BB // 16):
                    v = plsc.load_gather(rows_v.at[nb], [row_idx[bg], col_idx[d]])
                    t_v[nb, dt, di, pl.ds(bg * 16, 16)] = v

    fire_gather(0, 0)

    @pl.loop(0, _SEQ_LEN, step=2)
    def _pair(l0):
        for b in range(2):
            l = l0 + b

            @pl.when(l < _SEQ_LEN - 1)
            def _next():
                fire_gather(l + 1, 1 - b)

            drain_gather(b)

            @pl.when(l >= 2)
            def _reclaim():
                drain_write(b)

            transpose(b)
            fire_write(l, b)

    for b in range(2):
        drain_write(b)


@jax.jit
def kernel(x, table):
    xv = x.reshape(_NW, _BB, _LT, 8).transpose(2, 0, 3, 1).astype(jnp.int32)

    mesh = plsc.VectorSubcoreMesh(
        core_axis_name="c", subcore_axis_name="s",
        num_cores=_NC, num_subcores=_NS,
    )
    o5 = pl.kernel(
        _body,
        out_type=jax.ShapeDtypeStruct(
            (_SEQ_LEN, _DT, _NW, 8, _BB), jnp.float32
        ),
        mesh=mesh,
        scratch_types=[
            pltpu.VMEM((_LT, 8, _BB), jnp.int32),
            pltpu.VMEM((2, _BB, _EMBED_DIM), jnp.float32),
            pltpu.VMEM((2, _DT, 8, _BB), jnp.float32),
            pltpu.SemaphoreType.DMA((2,)),
            pltpu.SemaphoreType.DMA((2,)),
        ],
        compiler_params=pltpu.CompilerParams(
            use_tc_tiling_on_sc=False, needs_layout_passes=False
        ),
    )(xv, table)
    return o5.transpose(0, 2, 4, 1, 3).reshape(_SEQ_LEN, _BATCH, _EMBED_DIM)


# speedup vs baseline: 1.6542x; 1.6542x over previous
import jax
import jax.numpy as jnp
from jax import lax
from jax.experimental import pallas as pl
from jax.experimental.pallas import tpu as pltpu
from jax.experimental.pallas import tpu_sc as plsc

_EMBED_DIM = 32
_BATCH = 4096
_SEQ_LEN = 200

_NC = 2
_NS = 16
_NW = _NC * _NS
_BB = _BATCH // _NW
_LT = _SEQ_LEN // 8
_DT = _EMBED_DIM // 8


def _body(xv_hbm, table_hbm, o5_hbm, idxs, rows_v, t_v, gsem, wsem):
    w = lax.axis_index("s") * _NC + lax.axis_index("c")

    pltpu.sync_copy(xv_hbm.at[:, w], idxs)

    lane = lax.iota(jnp.int32, 16)
    dt_idx = [(lane + 16 * h) // 8 for h in range(2)]
    di_idx = [(lane + 16 * h) % 8 for h in range(2)]

    _ROW_BYTES = _BB * _EMBED_DIM * 4

    def fire_gather(l, nb):
        pltpu.async_copy(
            table_hbm.at[idxs.at[lax.div(l, 8), lax.rem(l, 8)]],
            rows_v.at[nb],
            gsem.at[nb],
        )

    def drain_gather(nb):
        pltpu.make_async_copy(
            table_hbm.at[pl.ds(0, _BB)], rows_v.at[nb], gsem.at[nb]
        ).wait()

    def fire_write(l, nb):
        pltpu.async_copy(
            t_v.at[nb].at[:, :, pl.ds(0, _BB)], o5_hbm.at[l].at[:, w], wsem.at[nb]
        )

    def drain_write(nb):
        pltpu.make_async_copy(
            t_v.at[nb].at[:, :, pl.ds(0, _BB)], o5_hbm.at[0].at[:, 0], wsem.at[nb]
        ).wait()

    def transpose(nb):
        for b in range(_BB):
            bi = jnp.full((16,), b, jnp.int32)
            for h in range(2):
                v = rows_v[nb, b, pl.ds(16 * h, 16)]
                plsc.store_scatter(t_v.at[nb], [dt_idx[h], di_idx[h], bi], v)

    fire_gather(0, 0)
    fire_gather(1, 1)
    for b in range(2):
        drain_gather(b)
        transpose(b)
        fire_gather(b + 2, b)
        fire_write(b, b)

    @pl.loop(1, (_SEQ_LEN - 4) // 2 + 1)
    def _pair(p):
        l0 = 2 * p
        for b in range(2):
            l = l0 + b
            drain_gather(b)
            drain_write(b)
            transpose(b)
            fire_gather(l + 2, b)
            fire_write(l, b)

    for b in range(2):
        l = _SEQ_LEN - 2 + b
        drain_gather(b)
        drain_write(b)
        transpose(b)
        fire_write(l, b)
    for b in range(2):
        drain_write(b)


@jax.jit
def kernel(x, table):
    xv = x.reshape(_NW, _BB, _LT, 8).transpose(2, 0, 3, 1).astype(jnp.int32)

    mesh = plsc.VectorSubcoreMesh(
        core_axis_name="c", subcore_axis_name="s",
        num_cores=_NC, num_subcores=_NS,
    )
    o5 = pl.kernel(
        _body,
        out_type=jax.ShapeDtypeStruct(
            (_SEQ_LEN, _DT, _NW, 8, _BB), jnp.float32
        ),
        mesh=mesh,
        scratch_types=[
            pltpu.VMEM((_LT, 8, _BB), jnp.int32),
            pltpu.VMEM((2, _BB, _EMBED_DIM), jnp.float32),
            pltpu.VMEM((2, _DT, 8, _BB + 1), jnp.float32),
            pltpu.SemaphoreType.DMA((2,)),
            pltpu.SemaphoreType.DMA((2,)),
        ],
        compiler_params=pltpu.CompilerParams(
            use_tc_tiling_on_sc=False, needs_layout_passes=False
        ),
    )(xv, table)
    return o5.transpose(0, 2, 4, 1, 3).reshape(_SEQ_LEN, _BATCH, _EMBED_DIM)
